# Initial kernel scaffold; baseline (speedup 1.0000x reference)
import functools

import jax
import jax.numpy as jnp
from jax import lax
from jax.experimental import pallas as pl
from jax.experimental.pallas import tpu as pltpu
from jax.experimental.pallas import tpu_sc as plsc

N_A = 131072
N_P = 32768
B = 512
D = 300

NC = 2
NS = 16
NW = NC * NS

G = 128
A_PER_W = N_A // NW
F_PER_W = N_P // NW
A_GRPS = A_PER_W // G
F_GRPS = F_PER_W // G
ROWS_PER_TILE = B // NS


def _sc_body(atom_h, frag_h, aseg_h, fseg_h, z2d_h, z1_h,
             part_o, cnt_o,
             acc, cnta, cntf, buf, aidx, fidx, ones, stage, zc,
             sem0, sem1):
    c = lax.axis_index("c")
    s = lax.axis_index("s")
    wid = s * NC + c

    for i in range(G // 16):
        ones[pl.ds(i * 16, 16)] = jnp.ones((16,), jnp.float32)

    pltpu.sync_copy(z2d_h.at[pl.ds(s * ROWS_PER_TILE, ROWS_PER_TILE)], stage)
    pltpu.sync_copy(stage, acc.at[pl.ds(s * ROWS_PER_TILE, ROWS_PER_TILE)])
    pltpu.sync_copy(z1_h.at[pl.ds(s * ROWS_PER_TILE, ROWS_PER_TILE)], zc)
    pltpu.sync_copy(zc, cnta.at[pl.ds(s * ROWS_PER_TILE, ROWS_PER_TILE)])
    pltpu.sync_copy(zc, cntf.at[pl.ds(s * ROWS_PER_TILE, ROWS_PER_TILE)])
    plsc.subcore_barrier()

    pltpu.sync_copy(aseg_h.at[pl.ds(wid * A_GRPS, A_GRPS)], aidx)
    pltpu.sync_copy(fseg_h.at[pl.ds(wid * F_GRPS, F_GRPS)], fidx)

    sems = (sem0, sem1)

    def run(src_h, idx, n_grps, base, cnt_ref):
        pltpu.async_copy(src_h.at[pl.ds(base, G)], buf.at[0], sems[0])
        for g in range(n_grps):
            cur = g & 1
            if g + 1 < n_grps:
                pltpu.async_copy(src_h.at[pl.ds(base + (g + 1) * G, G)],
                                 buf.at[1 - cur], sems[1 - cur])
            pltpu.make_async_copy(src_h.at[pl.ds(base + g * G, G)],
                                  buf.at[cur], sems[cur]).wait()
            pltpu.sync_copy(buf.at[cur], acc.at[idx.at[g]], add=True)
            pltpu.sync_copy(ones, cnt_ref.at[idx.at[g]], add=True)

    run(atom_h, aidx, A_GRPS, wid * A_PER_W, cnta)
    run(frag_h, fidx, F_GRPS, wid * F_PER_W, cntf)

    plsc.subcore_barrier()

    r0 = s * ROWS_PER_TILE
    pltpu.sync_copy(acc.at[pl.ds(r0, ROWS_PER_TILE)], stage)
    pltpu.sync_copy(stage, part_o.at[c].at[pl.ds(r0, ROWS_PER_TILE)])
    pltpu.sync_copy(cnta.at[pl.ds(r0, ROWS_PER_TILE)], zc)
    pltpu.sync_copy(zc, cnt_o.at[c, 0].at[pl.ds(r0, ROWS_PER_TILE)])
    pltpu.sync_copy(cntf.at[pl.ds(r0, ROWS_PER_TILE)], zc)
    pltpu.sync_copy(zc, cnt_o.at[c, 1].at[pl.ds(r0, ROWS_PER_TILE)])


_sc_call = pl.kernel(
    _sc_body,
    out_type=(
        jax.ShapeDtypeStruct((NC, B, D), jnp.float32),
        jax.ShapeDtypeStruct((NC, 2, B), jnp.float32),
    ),
    mesh=plsc.VectorSubcoreMesh(core_axis_name="c", subcore_axis_name="s"),
    scratch_types=[
        pltpu.VMEM_SHARED((B, D), jnp.float32),
        pltpu.VMEM_SHARED((B,), jnp.float32),
        pltpu.VMEM_SHARED((B,), jnp.float32),
        pltpu.VMEM((2, G, D), jnp.float32),
        pltpu.VMEM((A_GRPS, G), jnp.int32),
        pltpu.VMEM((F_GRPS, G), jnp.int32),
        pltpu.VMEM((G,), jnp.float32),
        pltpu.VMEM((ROWS_PER_TILE, D), jnp.float32),
        pltpu.VMEM((ROWS_PER_TILE,), jnp.float32),
        pltpu.SemaphoreType.DMA,
        pltpu.SemaphoreType.DMA,
    ],
)


def _combine_body(part_ref, cnt_ref, out_ref):
    p = part_ref[0] + part_ref[1]
    cs = cnt_ref[0] + cnt_ref[1]
    ma = jnp.max(cs[0:1, :])
    mf = jnp.max(cs[1:2, :])
    out_ref[...] = p * (1.0 / (ma + mf))


def kernel(atom_embed, frag_embed, atom_seg, frag_seg):
    aseg = atom_seg.astype(jnp.int32).reshape(N_A // G, G)
    fseg = frag_seg.astype(jnp.int32).reshape(N_P // G, G)
    z2d = jnp.zeros((B, D), jnp.float32)
    z1 = jnp.zeros((B,), jnp.float32)
    part, cnt = _sc_call(atom_embed, frag_embed, aseg, fseg, z2d, z1)
    return pl.pallas_call(
        _combine_body,
        out_shape=jax.ShapeDtypeStruct((B, D), jnp.float32),
    )(part, cnt)


# speedup vs baseline: 1.2719x; 1.2719x over previous
import functools

import jax
import jax.numpy as jnp
from jax import lax
from jax.experimental import pallas as pl
from jax.experimental.pallas import tpu as pltpu
from jax.experimental.pallas import tpu_sc as plsc

N_A = 131072
N_P = 32768
B = 512
D = 300

NC = 2
NS = 16
NW = NC * NS

G = 128
A_PER_W = N_A // NW
F_PER_W = N_P // NW
A_GRPS = A_PER_W // G
F_GRPS = F_PER_W // G
ROWS_PER_TILE = B // NS
DP = 304


def _sc_body(atom_h, frag_h, aseg_h, fseg_h, z2d_h, z1_h,
             part_o, cnt_o,
             acc, cnta, cntf, buf, aidx, fidx, ones, stage, zc,
             sem0, sem1):
    c = lax.axis_index("c")
    s = lax.axis_index("s")
    wid = s * NC + c

    for i in range(G // 16):
        ones[pl.ds(i * 16, 16)] = jnp.ones((16,), jnp.float32)

    pltpu.sync_copy(z2d_h.at[pl.ds(s * ROWS_PER_TILE, ROWS_PER_TILE)], stage)
    pltpu.sync_copy(stage, acc.at[pl.ds(s * ROWS_PER_TILE, ROWS_PER_TILE)])
    pltpu.sync_copy(z1_h.at[pl.ds(s * ROWS_PER_TILE, ROWS_PER_TILE)], zc)
    pltpu.sync_copy(zc, cnta.at[pl.ds(s * ROWS_PER_TILE, ROWS_PER_TILE)])
    pltpu.sync_copy(zc, cntf.at[pl.ds(s * ROWS_PER_TILE, ROWS_PER_TILE)])
    plsc.subcore_barrier()

    pltpu.sync_copy(aseg_h.at[pl.ds(wid * A_GRPS, A_GRPS)], aidx)
    pltpu.sync_copy(fseg_h.at[pl.ds(wid * F_GRPS, F_GRPS)], fidx)

    sems = (sem0, sem1)

    def run(src_h, idx, n_grps, base, cnt_ref):
        pltpu.async_copy(src_h.at[pl.ds(base, G)], buf.at[0], sems[0])
        for g in range(n_grps):
            cur = g & 1
            if g + 1 < n_grps:
                pltpu.async_copy(src_h.at[pl.ds(base + (g + 1) * G, G)],
                                 buf.at[1 - cur], sems[1 - cur])
            pltpu.make_async_copy(src_h.at[pl.ds(base + g * G, G)],
                                  buf.at[cur], sems[cur]).wait()
            pltpu.sync_copy(buf.at[cur], acc.at[idx.at[g]], add=True)
            pltpu.sync_copy(ones, cnt_ref.at[idx.at[g]], add=True)

    run(atom_h, aidx, A_GRPS, wid * A_PER_W, cnta)
    run(frag_h, fidx, F_GRPS, wid * F_PER_W, cntf)

    plsc.subcore_barrier()

    r0 = s * ROWS_PER_TILE
    pltpu.sync_copy(acc.at[pl.ds(r0, ROWS_PER_TILE)], stage)
    pltpu.sync_copy(stage, part_o.at[c].at[pl.ds(r0, ROWS_PER_TILE)])
    pltpu.sync_copy(cnta.at[pl.ds(r0, ROWS_PER_TILE)], zc)
    pltpu.sync_copy(zc, cnt_o.at[c, 0].at[pl.ds(r0, ROWS_PER_TILE)])
    pltpu.sync_copy(cntf.at[pl.ds(r0, ROWS_PER_TILE)], zc)
    pltpu.sync_copy(zc, cnt_o.at[c, 1].at[pl.ds(r0, ROWS_PER_TILE)])


_sc_call = pl.kernel(
    _sc_body,
    out_type=(
        jax.ShapeDtypeStruct((NC, B, DP), jnp.float32),
        jax.ShapeDtypeStruct((NC, 2, B), jnp.float32),
    ),
    mesh=plsc.VectorSubcoreMesh(core_axis_name="c", subcore_axis_name="s"),
    compiler_params=pltpu.CompilerParams(use_tc_tiling_on_sc=False),
    scratch_types=[
        pltpu.VMEM_SHARED((B, DP), jnp.float32),
        pltpu.VMEM_SHARED((B,), jnp.float32),
        pltpu.VMEM_SHARED((B,), jnp.float32),
        pltpu.VMEM((2, G, DP), jnp.float32),
        pltpu.VMEM((A_GRPS, G), jnp.int32),
        pltpu.VMEM((F_GRPS, G), jnp.int32),
        pltpu.VMEM((G,), jnp.float32),
        pltpu.VMEM((ROWS_PER_TILE, DP), jnp.float32),
        pltpu.VMEM((ROWS_PER_TILE,), jnp.float32),
        pltpu.SemaphoreType.DMA,
        pltpu.SemaphoreType.DMA,
    ],
)


def _combine_body(part_ref, cnt_ref, out_ref):
    p = part_ref[0, :, :D] + part_ref[1, :, :D]
    cs = cnt_ref[0] + cnt_ref[1]
    ma = jnp.max(cs[0:1, :])
    mf = jnp.max(cs[1:2, :])
    out_ref[...] = p * (1.0 / (ma + mf))


def kernel(atom_embed, frag_embed, atom_seg, frag_seg):
    atom_p = jnp.pad(atom_embed, ((0, 0), (0, DP - D)))
    frag_p = jnp.pad(frag_embed, ((0, 0), (0, DP - D)))
    aseg = atom_seg.astype(jnp.int32).reshape(N_A // G, G)
    fseg = frag_seg.astype(jnp.int32).reshape(N_P // G, G)
    z2d = jnp.zeros((B, DP), jnp.float32)
    z1 = jnp.zeros((B,), jnp.float32)
    part, cnt = _sc_call(atom_p, frag_p, aseg, fseg, z2d, z1)
    return pl.pallas_call(
        _combine_body,
        out_shape=jax.ShapeDtypeStruct((B, D), jnp.float32),
    )(part, cnt)
